# baseline (speedup 1.0000x reference)
import functools

import jax
import jax.numpy as jnp
from jax import lax
from jax.experimental import pallas as pl
from jax.experimental.pallas import tpu as pltpu
from jax.experimental.pallas import tpu_sc as plsc

N = 10000
E = 320000
NG = 64
D_IN = 128
D_H = 128
D_H2 = 32

NC = 2
NS = 16
NW = NC * NS
L = 16

OWN = 320
N_PAD = OWN * NW
NROW = N_PAD // NS
ZCH = 128
EPT = E // NW
CHUNK = 80
NCHUNK = EPT // CHUNK

BLK = 2000
NBLK = E // BLK
NGRP = BLK // L
CAPB = E + 80

_MESH = plsc.VectorSubcoreMesh(core_axis_name="c", subcore_axis_name="s")
NEG = float("-inf")

import dataclasses as _dc

_SC_PARAMS = pltpu.CompilerParams()
if "needs_layout_passes" in pltpu.CompilerParams.__dataclass_fields__:
    _SC_PARAMS = _dc.replace(_SC_PARAMS, needs_layout_passes=False)

_SC_PARAMS_U = _dc.replace(_SC_PARAMS, use_tc_tiling_on_sc=False)


def _wid():
    return lax.axis_index("s") * NC + lax.axis_index("c")


def _deg_body(dst3_hbm, out_hbm, didx2, ones_b, zbuf, acc_sh, sem):
    c = lax.axis_index("c")
    s = lax.axis_index("s")
    w = _wid()

    zrow = jnp.zeros((L,), jnp.float32)
    one = jnp.ones((L,), jnp.float32)

    @pl.loop(0, ZCH)
    def _(i):
        zbuf[i, pl.ds(0, L)] = zrow

    @pl.loop(0, CHUNK)
    def _(i):
        ones_b[i, pl.ds(0, L)] = one

    pltpu.sync_copy(dst3_hbm.at[w], didx2)
    for j in range(5):
        pltpu.sync_copy(zbuf, acc_sh.at[pl.ds(s * NROW + j * ZCH, ZCH), :])
    plsc.subcore_barrier()

    @pl.loop(0, NCHUNK // 25)
    def _(r):
        @pl.loop(0, 25)
        def _(j):
            pltpu.async_copy(ones_b, acc_sh.at[didx2.at[r * 25 + j]], sem,
                             add=True)

        @pl.loop(0, 25)
        def _(j):
            pltpu.make_async_copy(ones_b, acc_sh.at[didx2.at[r * 25 + j]],
                                  sem).wait()

    plsc.subcore_barrier()
    for j in range(5):
        r0 = s * NROW + j * ZCH
        pltpu.sync_copy(acc_sh.at[pl.ds(r0, ZCH), :],
                        out_hbm.at[c, pl.ds(r0, ZCH), :])


@jax.jit
def _deg_call(ei_dst3):
    return pl.kernel(
        _deg_body,
        out_type=jax.ShapeDtypeStruct((NC, N_PAD, L), jnp.float32),
        mesh=_MESH,
        compiler_params=_SC_PARAMS_U,
        scratch_types=[
            pltpu.VMEM((NCHUNK, CHUNK), jnp.int32),
            pltpu.VMEM((CHUNK, L), jnp.float32),
            pltpu.VMEM((ZCH, L), jnp.float32),
            pltpu.VMEM_SHARED((N_PAD, L), jnp.float32),
            pltpu.SemaphoreType.DMA,
        ],
    )(ei_dst3)


def _add_body(d, g_hbm, src_hbm, dst_hbm, out_hbm,
              sb0, sb1, sb2, db0, db1, db2, rows0, rows1, rows2,
              zbuf, acc_sh, gs0, gs1, gs2):
    c = lax.axis_index("c")
    s = lax.axis_index("s")
    w = _wid()
    sb = [sb0, sb1, sb2]
    db = [db0, db1, db2]
    rows = [rows0, rows1, rows2]
    gsem = [gs0, gs1, gs2]

    zrow = jnp.zeros((L,), jnp.float32)

    @pl.loop(0, 64)
    def _(i):
        for q in range(d // L):
            zbuf[i, pl.ds(q * L, L)] = zrow

    for j in range(10):
        pltpu.sync_copy(zbuf, acc_sh.at[pl.ds(s * NROW + j * 64, 64), :])
    plsc.subcore_barrier()

    def fetch(cc, p):
        be = w * EPT + cc * CHUNK
        pltpu.sync_copy(src_hbm.at[pl.ds(be, CHUNK)], sb[p])
        pltpu.sync_copy(dst_hbm.at[pl.ds(be, CHUNK)], db[p])
        pltpu.async_copy(g_hbm.at[sb[p]], rows[p], gsem[p])

    for p in range(3):
        fetch(p, p)

    @pl.loop(0, (NCHUNK + 2) // 3)
    def _(i):
        for p in range(3):
            cc = i * 3 + p

            @pl.when(cc < NCHUNK)
            def _(p=p, cc=cc):
                pltpu.make_async_copy(g_hbm.at[sb[p]], rows[p],
                                      gsem[p]).wait()
                pltpu.sync_copy(rows[p], acc_sh.at[db[p]], add=True)

            @pl.when(cc + 3 < NCHUNK)
            def _(p=p, cc=cc):
                fetch(cc + 3, p)

    plsc.subcore_barrier()
    for j in range(5):
        r0 = s * NROW + j * ZCH
        pltpu.sync_copy(acc_sh.at[pl.ds(r0, ZCH), :],
                        out_hbm.at[c, pl.ds(r0, ZCH), :])


def _make_add_call(d):
    @jax.jit
    def call(g, src, dst):
        return pl.kernel(
            functools.partial(_add_body, d),
            out_type=jax.ShapeDtypeStruct((NC, N_PAD, d), jnp.float32),
            mesh=_MESH,
            compiler_params=_SC_PARAMS if d % 128 == 0 else _SC_PARAMS_U,
            scratch_types=(
                [pltpu.VMEM((CHUNK,), jnp.int32)] * 6
                + [pltpu.VMEM((CHUNK, d), jnp.float32)] * 3
                + [pltpu.VMEM((64, d), jnp.float32),
                   pltpu.VMEM_SHARED((N_PAD, d), jnp.float32)]
                + [pltpu.SemaphoreType.DMA] * 3
            ),
        )(g, src, dst)

    return call


_add_call_128 = _make_add_call(D_H)
_add_call_32 = _make_add_call(D_H2)


def _bucket_body(asrc_hbm, adst_hbm, bsrc_hbm, bdst_hbm, cnt_hbm,
                 lsrc0, ldst0, lsrc1, ldst1, ssrc, sdst, cbuf,
                 la0, lb0, la1, lb1):
    w = _wid()
    base = w * OWN
    hi = base + OWN
    bb = w * CAPB

    lsrc = [lsrc0, lsrc1]
    ldst = [ldst0, ldst1]
    lsem = [la0, la1]
    dsem = [lb0, lb1]

    pltpu.sync_copy(asrc_hbm.at[pl.ds(0, BLK)], lsrc0)
    pltpu.sync_copy(adst_hbm.at[pl.ds(0, BLK)], ldst0)

    def process(blk, ls, ld, carry):
        off, nb = carry

        def grp(g, off):
            dv = ld[pl.ds(g * L, L)]
            sv = ls[pl.ds(g * L, L)]
            m = (dv >= base) & (dv < hi)
            plsc.store_compressed(sdst.at[pl.ds(off, L)], dv, mask=m)
            plsc.store_compressed(ssrc.at[pl.ds(off, L)], sv, mask=m)
            cntv = plsc.all_reduce_population_count(m)
            return off + jnp.squeeze(lax.slice(cntv, (0,), (1,)))

        off = lax.fori_loop(0, NGRP, grp, off, unroll=False)

        nunits = off // 80

        def flush(u, _):
            pltpu.sync_copy(ssrc.at[pl.ds(u * 80, 80)],
                            bsrc_hbm.at[pl.ds(bb + (nb + u) * 80, 80)])
            pltpu.sync_copy(sdst.at[pl.ds(u * 80, 80)],
                            bdst_hbm.at[pl.ds(bb + (nb + u) * 80, 80)])
            return 0

        lax.fori_loop(0, nunits, flush, 0, unroll=False)
        rem = off - nunits * 80
        for r in range(5):
            sv = ssrc[pl.ds(nunits * 80 + r * L, L)]
            dv = sdst[pl.ds(nunits * 80 + r * L, L)]
            ssrc[pl.ds(r * L, L)] = sv
            sdst[pl.ds(r * L, L)] = dv
        return (rem, nb + nunits)

    def rnd(i, carry):
        for p in range(2):
            blk = 2 * i + p

            @pl.when(blk + 1 < NBLK)
            def _():
                nxt = (blk + 1) * BLK
                pltpu.async_copy(asrc_hbm.at[pl.ds(nxt, BLK)], lsrc[1 - p],
                                 lsem[1 - p])
                pltpu.async_copy(adst_hbm.at[pl.ds(nxt, BLK)], ldst[1 - p],
                                 dsem[1 - p])

            @pl.when(blk > 0)
            def _():
                pltpu.make_async_copy(asrc_hbm.at[pl.ds(blk * BLK, BLK)],
                                      lsrc[p], lsem[p]).wait()
                pltpu.make_async_copy(adst_hbm.at[pl.ds(blk * BLK, BLK)],
                                      ldst[p], dsem[p]).wait()

            carry = process(blk, lsrc[p], ldst[p], carry)
        return carry

    off, nb = lax.fori_loop(0, NBLK // 2, rnd, (jnp.int32(0), jnp.int32(0)),
                            unroll=False)

    sent_s = jnp.full((L,), N, jnp.int32)
    sent_d = jnp.full((L,), base, jnp.int32)
    for r in range(5):
        ssrc[pl.ds(off + r * L, L)] = sent_s
        sdst[pl.ds(off + r * L, L)] = sent_d

    @pl.when(off > 0)
    def _():
        pltpu.sync_copy(ssrc.at[pl.ds(0, 80)],
                        bsrc_hbm.at[pl.ds(bb + nb * 80, 80)])
        pltpu.sync_copy(sdst.at[pl.ds(0, 80)],
                        bdst_hbm.at[pl.ds(bb + nb * 80, 80)])

    nbf = nb + (off > 0).astype(jnp.int32)
    cbuf[pl.ds(0, L)] = jnp.full((L,), 0, jnp.int32) + nbf
    pltpu.sync_copy(cbuf, cnt_hbm.at[pl.ds(w * L, L)])


@jax.jit
def _bucket_call(aj_src, aj_dst):
    return pl.kernel(
        _bucket_body,
        out_type=(
            jax.ShapeDtypeStruct((NW * CAPB,), jnp.int32),
            jax.ShapeDtypeStruct((NW * CAPB,), jnp.int32),
            jax.ShapeDtypeStruct((NW * L,), jnp.int32),
        ),
        mesh=_MESH,
        compiler_params=_SC_PARAMS,
        scratch_types=(
            [pltpu.VMEM((BLK,), jnp.int32)] * 4
            + [pltpu.VMEM((BLK + 96,), jnp.int32)] * 2
            + [pltpu.VMEM((L,), jnp.int32)]
            + [pltpu.SemaphoreType.DMA] * 4
        ),
    )(aj_src, aj_dst)


def _max_body(d, tab_hbm, bsrc_hbm, bdst_hbm, cnt_hbm, out_hbm,
              sb0, sb1, sb2, sb3, sb4, db0, db1, db2, db3, db4,
              rows0, rows1, rows2, rows3, rows4,
              acc, cbuf, gs0, gs1, gs2, gs3, gs4):
    w = _wid()
    base = w * OWN
    bb = w * CAPB

    sb = [sb0, sb1, sb2, sb3, sb4]
    db = [db0, db1, db2, db3, db4]
    rows = [rows0, rows1, rows2, rows3, rows4]
    gsem = [gs0, gs1, gs2, gs3, gs4]

    ninf = jnp.full((L,), NEG, jnp.float32)

    @pl.loop(0, OWN * d // L)
    def _(i):
        acc[pl.ds(i * L, L)] = ninf

    pltpu.sync_copy(cnt_hbm.at[pl.ds(w * L, L)], cbuf)
    nb = jnp.max(cbuf[pl.ds(0, L)], axis=0)

    def fetch(cc, p):
        pltpu.sync_copy(bsrc_hbm.at[pl.ds(bb + cc * 80, 80)], sb[p])
        pltpu.sync_copy(bdst_hbm.at[pl.ds(bb + cc * 80, 80)], db[p])
        pltpu.async_copy(tab_hbm.at[sb[p]], rows[p], gsem[p])

    for p in range(5):
        @pl.when(p < nb)
        def _(p=p):
            fetch(p, p)

    def rmw(p, cc):
        pltpu.make_async_copy(tab_hbm.at[sb[p]], rows[p], gsem[p]).wait()

        def grp(g, _):
            offv = (db[p][pl.ds(g * L, L)] - base) * d
            for k in range(L):
                dlo = jnp.squeeze(lax.slice(offv, (k,), (k + 1,)))
                e = g * L + k
                for q in range(d // L):
                    a = acc[pl.ds(dlo + q * L, L)]
                    r = rows[p][e, pl.ds(q * L, L)]
                    acc[pl.ds(dlo + q * L, L)] = jnp.maximum(a, r)
            return 0

        lax.fori_loop(0, 5, grp, 0, unroll=False)

    def rnd(i, _):
        for p in range(5):
            cc = i * 5 + p

            @pl.when(cc < nb)
            def _(p=p, cc=cc):
                rmw(p, cc)

            @pl.when(cc + 5 < nb)
            def _(p=p, cc=cc):
                fetch(cc + 5, p)
        return 0

    lax.fori_loop(0, (nb + 4) // 5, rnd, 0, unroll=False)

    pltpu.sync_copy(acc, out_hbm.at[pl.ds(w * OWN * d, OWN * d)])


def _make_max_call(d):
    @jax.jit
    def call(tab, bsrc, bdst, bcnt):
        flat = pl.kernel(
            functools.partial(_max_body, d),
            out_type=jax.ShapeDtypeStruct((N_PAD * d,), jnp.float32),
            mesh=_MESH,
            compiler_params=_SC_PARAMS if d % 128 == 0 else _SC_PARAMS_U,
            scratch_types=(
                [pltpu.VMEM((80,), jnp.int32)] * 10
                + [pltpu.VMEM((80, d), jnp.float32)] * 5
                + [pltpu.VMEM((OWN * d,), jnp.float32),
                   pltpu.VMEM((L,), jnp.int32)]
                + [pltpu.SemaphoreType.DMA] * 5
            ),
        )(tab, bsrc, bdst, bcnt)
        return flat.reshape(N_PAD, d)

    return call


_max_call_128 = _make_max_call(D_H)
_max_call_32 = _make_max_call(D_H2)


def _dinv(deg_ref):
    total = 1.0 + deg_ref[0, :, 0:1] + deg_ref[1, :, 0:1]
    return lax.rsqrt(total)


def _tc1_body(x_ref, w1_ref, deg_ref, g_ref):
    h = jnp.dot(x_ref[...], w1_ref[...], preferred_element_type=jnp.float32)
    g_ref[...] = h * _dinv(deg_ref)


@jax.jit
def _tc1(x, W1, deg):
    return pl.pallas_call(
        _tc1_body,
        out_shape=jax.ShapeDtypeStruct((N, D_H), jnp.float32),
    )(x, W1, deg)


def _tc_mid_body(dh, p_ref, g_ref, deg_ref, we_ref, be_ref, b_ref,
                 a_ref, bv_ref):
    dinv = _dinv(deg_ref)
    h = jax.nn.relu(dinv * (p_ref[0] + p_ref[1] + g_ref[...]) + b_ref[...])
    wt = we_ref[0:dh, :]
    wb = we_ref[dh:2 * dh, :]
    a_ref[...] = jnp.dot(h, wt - wb, preferred_element_type=jnp.float32) \
        + be_ref[...]
    bv_ref[...] = jnp.dot(h, wb, preferred_element_type=jnp.float32)


def _make_tc_mid(dh):
    @jax.jit
    def call(p, g, deg, We, be, b):
        return pl.pallas_call(
            functools.partial(_tc_mid_body, dh),
            out_shape=(
                jax.ShapeDtypeStruct((N, dh), jnp.float32),
                jax.ShapeDtypeStruct((N, dh), jnp.float32),
            ),
        )(p, g, deg, We, be, b)

    return call


_tc2 = _make_tc_mid(D_H)
_tc4 = _make_tc_mid(D_H2)


def _tc3_body(m_ref, a_ref, deg_ref, w2_ref, g_ref):
    m = m_ref[...]
    h2 = jnp.where(m == NEG, 0.0, jax.nn.relu(a_ref[...] + m))
    g_ref[...] = jnp.dot(h2, w2_ref[...],
                         preferred_element_type=jnp.float32) * _dinv(deg_ref)


@jax.jit
def _tc3(m1, a1, deg, W2):
    return pl.pallas_call(
        _tc3_body,
        out_shape=jax.ShapeDtypeStruct((N, D_H2), jnp.float32),
    )(m1, a1, deg, W2)


def _tc5_body(m_ref, a_ref, wfc_ref, bfc_ref, batch_ref, out_ref):
    m = m_ref[...]
    h4 = jnp.where(m == NEG, 0.0, jax.nn.relu(a_ref[...] + m))
    y = jnp.dot(h4, wfc_ref[...], preferred_element_type=jnp.float32) \
        + bfc_ref[...]
    gid = batch_ref[...]
    onehot = (gid == lax.broadcasted_iota(jnp.int32, (1, NG), 1))
    onehot = onehot.astype(jnp.float32)
    s = jnp.sum(onehot * y, axis=0)
    cnt = jnp.sum(onehot, axis=0)
    out_ref[...] = (s / jnp.maximum(cnt, 1.0))[:, None]


@jax.jit
def _tc5(m2, a2, Wfc, bfc, batch2d):
    return pl.pallas_call(
        _tc5_body,
        out_shape=jax.ShapeDtypeStruct((NG, 1), jnp.float32),
    )(m2, a2, Wfc, bfc, batch2d)


@jax.jit
def kernel(x, edge_index, adj, batch, W1, b1, We1, be1, W2, b2, We2, be2,
           Wfc, bfc):
    ei_src = edge_index[0]
    ei_dst = edge_index[1]
    ei_dst3 = ei_dst.reshape(NW, NCHUNK, CHUNK)
    aj_src = adj[0]
    aj_dst = adj[1]

    deg = _deg_call(ei_dst3)[:, :N]
    bsrc, bdst, bcnt = _bucket_call(aj_src, aj_dst)

    g1 = _tc1(x, W1, deg)
    p1 = _add_call_128(g1, ei_src, ei_dst)[:, :N]
    a1, bv1 = _tc2(p1, g1, deg, We1, be1, b1)

    tab1 = jnp.concatenate([bv1, jnp.full((1, D_H), NEG, jnp.float32)], 0)
    m1 = _max_call_128(tab1, bsrc, bdst, bcnt)[:N]

    g2 = _tc3(m1, a1, deg, W2)
    p2 = _add_call_32(g2, ei_src, ei_dst)[:, :N]
    a2, bv2 = _tc4(p2, g2, deg, We2, be2, b2)

    tab2 = jnp.concatenate([bv2, jnp.full((1, D_H2), NEG, jnp.float32)], 0)
    m2 = _max_call_32(tab2, bsrc, bdst, bcnt)[:N]

    return _tc5(m2, a2, Wfc, bfc, batch.reshape(N, 1))


# speedup vs baseline: 1.1743x; 1.1743x over previous
import functools

import jax
import jax.numpy as jnp
from jax import lax
from jax.experimental import pallas as pl
from jax.experimental.pallas import tpu as pltpu
from jax.experimental.pallas import tpu_sc as plsc

N = 10000
E = 320000
NG = 64
D_IN = 128
D_H = 128
D_H2 = 32

NC = 2
NS = 16
NW = NC * NS
L = 16

OWN = 320
N_PAD = OWN * NW
NROW = N_PAD // NS
ZCH = 128
EPT = E // NW
CHUNK = 80
NCHUNK = EPT // CHUNK

BLK = 2000
NBLK = E // BLK
NGRP = BLK // L
CAPB = E + 80

_MESH = plsc.VectorSubcoreMesh(core_axis_name="c", subcore_axis_name="s")
NEG = float("-inf")

import dataclasses as _dc

_SC_PARAMS = pltpu.CompilerParams()
if "needs_layout_passes" in pltpu.CompilerParams.__dataclass_fields__:
    _SC_PARAMS = _dc.replace(_SC_PARAMS, needs_layout_passes=False)

_SC_PARAMS_U = _dc.replace(_SC_PARAMS, use_tc_tiling_on_sc=False)


def _wid():
    return lax.axis_index("s") * NC + lax.axis_index("c")


def _deg_body(dst3_hbm, out_hbm, didx2, ones_b, zbuf, acc_sh, sem):
    c = lax.axis_index("c")
    s = lax.axis_index("s")
    w = _wid()

    zrow = jnp.zeros((L,), jnp.float32)
    one = jnp.ones((L,), jnp.float32)

    @pl.loop(0, ZCH)
    def _(i):
        zbuf[i, pl.ds(0, L)] = zrow

    @pl.loop(0, CHUNK)
    def _(i):
        ones_b[i, pl.ds(0, L)] = one

    pltpu.sync_copy(dst3_hbm.at[w], didx2)
    for j in range(5):
        pltpu.sync_copy(zbuf, acc_sh.at[pl.ds(s * NROW + j * ZCH, ZCH), :])
    plsc.subcore_barrier()

    @pl.loop(0, NCHUNK // 25)
    def _(r):
        @pl.loop(0, 25)
        def _(j):
            pltpu.async_copy(ones_b, acc_sh.at[didx2.at[r * 25 + j]], sem,
                             add=True)

        @pl.loop(0, 25)
        def _(j):
            pltpu.make_async_copy(ones_b, acc_sh.at[didx2.at[r * 25 + j]],
                                  sem).wait()

    plsc.subcore_barrier()
    for j in range(5):
        r0 = s * NROW + j * ZCH
        pltpu.sync_copy(acc_sh.at[pl.ds(r0, ZCH), :],
                        out_hbm.at[c, pl.ds(r0, ZCH), :])


@jax.jit
def _deg_call(ei_dst3):
    return pl.kernel(
        _deg_body,
        out_type=jax.ShapeDtypeStruct((NC, N_PAD, L), jnp.float32),
        mesh=_MESH,
        compiler_params=_SC_PARAMS_U,
        scratch_types=[
            pltpu.VMEM((NCHUNK, CHUNK), jnp.int32),
            pltpu.VMEM((CHUNK, L), jnp.float32),
            pltpu.VMEM((ZCH, L), jnp.float32),
            pltpu.VMEM_SHARED((N_PAD, L), jnp.float32),
            pltpu.SemaphoreType.DMA,
        ],
    )(ei_dst3)


def _add_body(d, g_hbm, src_hbm, dst_hbm, out_hbm,
              sb0, sb1, sb2, db0, db1, db2, rows0, rows1, rows2,
              zbuf, acc_sh, gs0, gs1, gs2):
    c = lax.axis_index("c")
    s = lax.axis_index("s")
    w = _wid()
    sb = [sb0, sb1, sb2]
    db = [db0, db1, db2]
    rows = [rows0, rows1, rows2]
    gsem = [gs0, gs1, gs2]

    zrow = jnp.zeros((L,), jnp.float32)

    @pl.loop(0, 64)
    def _(i):
        for q in range(d // L):
            zbuf[i, pl.ds(q * L, L)] = zrow

    for j in range(10):
        pltpu.sync_copy(zbuf, acc_sh.at[pl.ds(s * NROW + j * 64, 64), :])
    plsc.subcore_barrier()

    def fetch(cc, p):
        be = w * EPT + cc * CHUNK
        pltpu.sync_copy(src_hbm.at[pl.ds(be, CHUNK)], sb[p])
        pltpu.sync_copy(dst_hbm.at[pl.ds(be, CHUNK)], db[p])
        pltpu.async_copy(g_hbm.at[sb[p]], rows[p], gsem[p])

    for p in range(3):
        fetch(p, p)

    @pl.loop(0, (NCHUNK + 2) // 3)
    def _(i):
        for p in range(3):
            cc = i * 3 + p

            @pl.when(cc < NCHUNK)
            def _(p=p, cc=cc):
                pltpu.make_async_copy(g_hbm.at[sb[p]], rows[p],
                                      gsem[p]).wait()
                pltpu.sync_copy(rows[p], acc_sh.at[db[p]], add=True)

            @pl.when(cc + 3 < NCHUNK)
            def _(p=p, cc=cc):
                fetch(cc + 3, p)

    plsc.subcore_barrier()
    for j in range(5):
        r0 = s * NROW + j * ZCH
        pltpu.sync_copy(acc_sh.at[pl.ds(r0, ZCH), :],
                        out_hbm.at[c, pl.ds(r0, ZCH), :])


def _make_add_call(d):
    @jax.jit
    def call(g, src, dst):
        return pl.kernel(
            functools.partial(_add_body, d),
            out_type=jax.ShapeDtypeStruct((NC, N_PAD, d), jnp.float32),
            mesh=_MESH,
            compiler_params=_SC_PARAMS if d % 128 == 0 else _SC_PARAMS_U,
            scratch_types=(
                [pltpu.VMEM((CHUNK,), jnp.int32)] * 6
                + [pltpu.VMEM((CHUNK, d), jnp.float32)] * 3
                + [pltpu.VMEM((64, d), jnp.float32),
                   pltpu.VMEM_SHARED((N_PAD, d), jnp.float32)]
                + [pltpu.SemaphoreType.DMA] * 3
            ),
        )(g, src, dst)

    return call


_add_call_128 = _make_add_call(D_H)
_add_call_32 = _make_add_call(D_H2)


def _bucket_body(asrc_hbm, adst_hbm, bsrc_hbm, bdst_hbm, cnt_hbm,
                 lsrc0, ldst0, lsrc1, ldst1, ssrc, sdst, cbuf,
                 la0, lb0, la1, lb1):
    w = _wid()
    base = w * OWN
    hi = base + OWN
    bb = w * CAPB

    lsrc = [lsrc0, lsrc1]
    ldst = [ldst0, ldst1]
    lsem = [la0, la1]
    dsem = [lb0, lb1]

    pltpu.sync_copy(asrc_hbm.at[pl.ds(0, BLK)], lsrc0)
    pltpu.sync_copy(adst_hbm.at[pl.ds(0, BLK)], ldst0)

    def process(blk, ls, ld, carry):
        off, nb = carry

        def grp(g, off):
            dv = ld[pl.ds(g * L, L)]
            sv = ls[pl.ds(g * L, L)]
            m = (dv >= base) & (dv < hi)
            plsc.store_compressed(sdst.at[pl.ds(off, L)], dv, mask=m)
            plsc.store_compressed(ssrc.at[pl.ds(off, L)], sv, mask=m)
            cntv = plsc.all_reduce_population_count(m)
            return off + jnp.squeeze(lax.slice(cntv, (0,), (1,)))

        off = lax.fori_loop(0, NGRP, grp, off, unroll=False)

        nunits = off // 80

        def flush(u, _):
            pltpu.sync_copy(ssrc.at[pl.ds(u * 80, 80)],
                            bsrc_hbm.at[pl.ds(bb + (nb + u) * 80, 80)])
            pltpu.sync_copy(sdst.at[pl.ds(u * 80, 80)],
                            bdst_hbm.at[pl.ds(bb + (nb + u) * 80, 80)])
            return 0

        lax.fori_loop(0, nunits, flush, 0, unroll=False)
        rem = off - nunits * 80
        for r in range(5):
            sv = ssrc[pl.ds(nunits * 80 + r * L, L)]
            dv = sdst[pl.ds(nunits * 80 + r * L, L)]
            ssrc[pl.ds(r * L, L)] = sv
            sdst[pl.ds(r * L, L)] = dv
        return (rem, nb + nunits)

    def rnd(i, carry):
        for p in range(2):
            blk = 2 * i + p

            @pl.when(blk + 1 < NBLK)
            def _():
                nxt = (blk + 1) * BLK
                pltpu.async_copy(asrc_hbm.at[pl.ds(nxt, BLK)], lsrc[1 - p],
                                 lsem[1 - p])
                pltpu.async_copy(adst_hbm.at[pl.ds(nxt, BLK)], ldst[1 - p],
                                 dsem[1 - p])

            @pl.when(blk > 0)
            def _():
                pltpu.make_async_copy(asrc_hbm.at[pl.ds(blk * BLK, BLK)],
                                      lsrc[p], lsem[p]).wait()
                pltpu.make_async_copy(adst_hbm.at[pl.ds(blk * BLK, BLK)],
                                      ldst[p], dsem[p]).wait()

            carry = process(blk, lsrc[p], ldst[p], carry)
        return carry

    off, nb = lax.fori_loop(0, NBLK // 2, rnd, (jnp.int32(0), jnp.int32(0)),
                            unroll=False)

    sent_s = jnp.full((L,), N, jnp.int32)
    sent_d = jnp.full((L,), base, jnp.int32)
    for r in range(5):
        ssrc[pl.ds(off + r * L, L)] = sent_s
        sdst[pl.ds(off + r * L, L)] = sent_d

    @pl.when(off > 0)
    def _():
        pltpu.sync_copy(ssrc.at[pl.ds(0, 80)],
                        bsrc_hbm.at[pl.ds(bb + nb * 80, 80)])
        pltpu.sync_copy(sdst.at[pl.ds(0, 80)],
                        bdst_hbm.at[pl.ds(bb + nb * 80, 80)])

    nbf = nb + (off > 0).astype(jnp.int32)
    cbuf[pl.ds(0, L)] = jnp.full((L,), 0, jnp.int32) + nbf
    pltpu.sync_copy(cbuf, cnt_hbm.at[pl.ds(w * L, L)])


@jax.jit
def _bucket_call(aj_src, aj_dst):
    return pl.kernel(
        _bucket_body,
        out_type=(
            jax.ShapeDtypeStruct((NW * CAPB,), jnp.int32),
            jax.ShapeDtypeStruct((NW * CAPB,), jnp.int32),
            jax.ShapeDtypeStruct((NW * L,), jnp.int32),
        ),
        mesh=_MESH,
        compiler_params=_SC_PARAMS,
        scratch_types=(
            [pltpu.VMEM((BLK,), jnp.int32)] * 4
            + [pltpu.VMEM((BLK + 96,), jnp.int32)] * 2
            + [pltpu.VMEM((L,), jnp.int32)]
            + [pltpu.SemaphoreType.DMA] * 4
        ),
    )(aj_src, aj_dst)


def _max_body(d, tab_hbm, bsrc_hbm, bdst_hbm, cnt_hbm, out_hbm,
              sb0, sb1, sb2, sb3, sb4, db0, db1, db2, db3, db4,
              rows0, rows1, rows2, rows3, rows4,
              acc, cbuf, gs0, gs1, gs2, gs3, gs4):
    w = _wid()
    base = w * OWN
    bb = w * CAPB

    sb = [sb0, sb1, sb2, sb3, sb4]
    db = [db0, db1, db2, db3, db4]
    rows = [rows0, rows1, rows2, rows3, rows4]
    gsem = [gs0, gs1, gs2, gs3, gs4]

    ninf = jnp.full((L,), NEG, jnp.float32)

    @pl.loop(0, OWN * d // L)
    def _(i):
        acc[pl.ds(i * L, L)] = ninf

    pltpu.sync_copy(cnt_hbm.at[pl.ds(w * L, L)], cbuf)
    nb = jnp.max(cbuf[pl.ds(0, L)], axis=0)

    def fetch(cc, p):
        pltpu.sync_copy(bsrc_hbm.at[pl.ds(bb + cc * 80, 80)], sb[p])
        pltpu.sync_copy(bdst_hbm.at[pl.ds(bb + cc * 80, 80)], db[p])
        pltpu.async_copy(tab_hbm.at[sb[p]], rows[p], gsem[p])

    for p in range(5):
        @pl.when(p < nb)
        def _(p=p):
            fetch(p, p)

    def rmw(p, cc):
        pltpu.make_async_copy(tab_hbm.at[sb[p]], rows[p], gsem[p]).wait()

        def grp(g, _):
            offv = (db[p][pl.ds(g * L, L)] - base) * d
            dlos = [jnp.squeeze(lax.slice(offv, (k,), (k + 1,)))
                    for k in range(L)]
            for k in range(L):
                dlo = dlos[k]
                e = g * L + k
                avs = [acc[pl.ds(dlo + q * L, L)] for q in range(d // L)]
                rvs = [rows[p][e, pl.ds(q * L, L)] for q in range(d // L)]
                for q in range(d // L):
                    acc[pl.ds(dlo + q * L, L)] = jnp.maximum(avs[q], rvs[q])
            return 0

        lax.fori_loop(0, 5, grp, 0, unroll=False)

    def rnd(i, _):
        for p in range(5):
            cc = i * 5 + p

            @pl.when(cc < nb)
            def _(p=p, cc=cc):
                rmw(p, cc)

            @pl.when(cc + 5 < nb)
            def _(p=p, cc=cc):
                fetch(cc + 5, p)
        return 0

    lax.fori_loop(0, (nb + 4) // 5, rnd, 0, unroll=False)

    pltpu.sync_copy(acc, out_hbm.at[pl.ds(w * OWN * d, OWN * d)])


def _make_max_call(d):
    @jax.jit
    def call(tab, bsrc, bdst, bcnt):
        flat = pl.kernel(
            functools.partial(_max_body, d),
            out_type=jax.ShapeDtypeStruct((N_PAD * d,), jnp.float32),
            mesh=_MESH,
            compiler_params=_SC_PARAMS if d % 128 == 0 else _SC_PARAMS_U,
            scratch_types=(
                [pltpu.VMEM((80,), jnp.int32)] * 10
                + [pltpu.VMEM((80, d), jnp.float32)] * 5
                + [pltpu.VMEM((OWN * d,), jnp.float32),
                   pltpu.VMEM((L,), jnp.int32)]
                + [pltpu.SemaphoreType.DMA] * 5
            ),
        )(tab, bsrc, bdst, bcnt)
        return flat.reshape(N_PAD, d)

    return call


_max_call_128 = _make_max_call(D_H)
_max_call_32 = _make_max_call(D_H2)


def _dinv(deg_ref):
    total = 1.0 + deg_ref[0, :, 0:1] + deg_ref[1, :, 0:1]
    return lax.rsqrt(total)


def _tc1_body(x_ref, w1_ref, deg_ref, g_ref):
    h = jnp.dot(x_ref[...], w1_ref[...], preferred_element_type=jnp.float32)
    g_ref[...] = h * _dinv(deg_ref)


@jax.jit
def _tc1(x, W1, deg):
    return pl.pallas_call(
        _tc1_body,
        out_shape=jax.ShapeDtypeStruct((N, D_H), jnp.float32),
    )(x, W1, deg)


def _tc_mid_body(dh, p_ref, g_ref, deg_ref, we_ref, be_ref, b_ref,
                 a_ref, bv_ref):
    dinv = _dinv(deg_ref)
    h = jax.nn.relu(dinv * (p_ref[0] + p_ref[1] + g_ref[...]) + b_ref[...])
    wt = we_ref[0:dh, :]
    wb = we_ref[dh:2 * dh, :]
    a_ref[...] = jnp.dot(h, wt - wb, preferred_element_type=jnp.float32) \
        + be_ref[...]
    bv_ref[...] = jnp.dot(h, wb, preferred_element_type=jnp.float32)


def _make_tc_mid(dh):
    @jax.jit
    def call(p, g, deg, We, be, b):
        return pl.pallas_call(
            functools.partial(_tc_mid_body, dh),
            out_shape=(
                jax.ShapeDtypeStruct((N, dh), jnp.float32),
                jax.ShapeDtypeStruct((N, dh), jnp.float32),
            ),
        )(p, g, deg, We, be, b)

    return call


_tc2 = _make_tc_mid(D_H)
_tc4 = _make_tc_mid(D_H2)


def _tc3_body(m_ref, a_ref, deg_ref, w2_ref, g_ref):
    m = m_ref[...]
    h2 = jnp.where(m == NEG, 0.0, jax.nn.relu(a_ref[...] + m))
    g_ref[...] = jnp.dot(h2, w2_ref[...],
                         preferred_element_type=jnp.float32) * _dinv(deg_ref)


@jax.jit
def _tc3(m1, a1, deg, W2):
    return pl.pallas_call(
        _tc3_body,
        out_shape=jax.ShapeDtypeStruct((N, D_H2), jnp.float32),
    )(m1, a1, deg, W2)


def _tc5_body(m_ref, a_ref, wfc_ref, bfc_ref, batch_ref, out_ref):
    m = m_ref[...]
    h4 = jnp.where(m == NEG, 0.0, jax.nn.relu(a_ref[...] + m))
    y = jnp.dot(h4, wfc_ref[...], preferred_element_type=jnp.float32) \
        + bfc_ref[...]
    gid = batch_ref[...]
    onehot = (gid == lax.broadcasted_iota(jnp.int32, (1, NG), 1))
    onehot = onehot.astype(jnp.float32)
    s = jnp.sum(onehot * y, axis=0)
    cnt = jnp.sum(onehot, axis=0)
    out_ref[...] = (s / jnp.maximum(cnt, 1.0))[:, None]


@jax.jit
def _tc5(m2, a2, Wfc, bfc, batch2d):
    return pl.pallas_call(
        _tc5_body,
        out_shape=jax.ShapeDtypeStruct((NG, 1), jnp.float32),
    )(m2, a2, Wfc, bfc, batch2d)


@jax.jit
def kernel(x, edge_index, adj, batch, W1, b1, We1, be1, W2, b2, We2, be2,
           Wfc, bfc):
    ei_src = edge_index[0]
    ei_dst = edge_index[1]
    ei_dst3 = ei_dst.reshape(NW, NCHUNK, CHUNK)
    aj_src = adj[0]
    aj_dst = adj[1]

    deg = _deg_call(ei_dst3)[:, :N]
    bsrc, bdst, bcnt = _bucket_call(aj_src, aj_dst)

    g1 = _tc1(x, W1, deg)
    p1 = _add_call_128(g1, ei_src, ei_dst)[:, :N]
    a1, bv1 = _tc2(p1, g1, deg, We1, be1, b1)

    tab1 = jnp.concatenate([bv1, jnp.full((1, D_H), NEG, jnp.float32)], 0)
    m1 = _max_call_128(tab1, bsrc, bdst, bcnt)[:N]

    g2 = _tc3(m1, a1, deg, W2)
    p2 = _add_call_32(g2, ei_src, ei_dst)[:, :N]
    a2, bv2 = _tc4(p2, g2, deg, We2, be2, b2)

    tab2 = jnp.concatenate([bv2, jnp.full((1, D_H2), NEG, jnp.float32)], 0)
    m2 = _max_call_32(tab2, bsrc, bdst, bcnt)[:N]

    return _tc5(m2, a2, Wfc, bfc, batch.reshape(N, 1))


# speedup vs baseline: 1.3904x; 1.1840x over previous
import functools

import jax
import jax.numpy as jnp
from jax import lax
from jax.experimental import pallas as pl
from jax.experimental.pallas import tpu as pltpu
from jax.experimental.pallas import tpu_sc as plsc

N = 10000
E = 320000
NG = 64
D_IN = 128
D_H = 128
D_H2 = 32

NC = 2
NS = 16
NW = NC * NS
L = 16

OWN = 320
N_PAD = OWN * NW
NROW = N_PAD // NS
ZCH = 128
EPT = E // NW
CHUNK = 80
NCHUNK = EPT // CHUNK

BLK = 2000
NBLK = E // BLK
NGRP = BLK // L
CAPB = E + 80

_MESH = plsc.VectorSubcoreMesh(core_axis_name="c", subcore_axis_name="s")
NEG = float("-inf")

import dataclasses as _dc

_SC_PARAMS = pltpu.CompilerParams()
if "needs_layout_passes" in pltpu.CompilerParams.__dataclass_fields__:
    _SC_PARAMS = _dc.replace(_SC_PARAMS, needs_layout_passes=False)

_SC_PARAMS_U = _dc.replace(_SC_PARAMS, use_tc_tiling_on_sc=False)


def _wid():
    return lax.axis_index("s") * NC + lax.axis_index("c")


def _deg_body(dst3_hbm, out_hbm, didx2, ones_b, zbuf, acc_sh, sem):
    c = lax.axis_index("c")
    s = lax.axis_index("s")
    w = _wid()

    zrow = jnp.zeros((L,), jnp.float32)
    one = jnp.ones((L,), jnp.float32)

    @pl.loop(0, ZCH)
    def _(i):
        zbuf[i, pl.ds(0, L)] = zrow

    @pl.loop(0, CHUNK)
    def _(i):
        ones_b[i, pl.ds(0, L)] = one

    pltpu.sync_copy(dst3_hbm.at[w], didx2)
    for j in range(5):
        pltpu.sync_copy(zbuf, acc_sh.at[pl.ds(s * NROW + j * ZCH, ZCH), :])
    plsc.subcore_barrier()

    @pl.loop(0, NCHUNK // 25)
    def _(r):
        @pl.loop(0, 25)
        def _(j):
            pltpu.async_copy(ones_b, acc_sh.at[didx2.at[r * 25 + j]], sem,
                             add=True)

        @pl.loop(0, 25)
        def _(j):
            pltpu.make_async_copy(ones_b, acc_sh.at[didx2.at[r * 25 + j]],
                                  sem).wait()

    plsc.subcore_barrier()
    for j in range(5):
        r0 = s * NROW + j * ZCH
        pltpu.sync_copy(acc_sh.at[pl.ds(r0, ZCH), :],
                        out_hbm.at[c, pl.ds(r0, ZCH), :])


@jax.jit
def _deg_call(ei_dst3):
    return pl.kernel(
        _deg_body,
        out_type=jax.ShapeDtypeStruct((NC, N_PAD, L), jnp.float32),
        mesh=_MESH,
        compiler_params=_SC_PARAMS_U,
        scratch_types=[
            pltpu.VMEM((NCHUNK, CHUNK), jnp.int32),
            pltpu.VMEM((CHUNK, L), jnp.float32),
            pltpu.VMEM((ZCH, L), jnp.float32),
            pltpu.VMEM_SHARED((N_PAD, L), jnp.float32),
            pltpu.SemaphoreType.DMA,
        ],
    )(ei_dst3)


def _add_body(d, g_hbm, src_hbm, dst_hbm, out_hbm,
              sb0, sb1, sb2, db0, db1, db2, rows0, rows1, rows2,
              zbuf, acc_sh, gs0, gs1, gs2, isem, jsem):
    c = lax.axis_index("c")
    s = lax.axis_index("s")
    w = _wid()
    sb = [sb0, sb1, sb2]
    db = [db0, db1, db2]
    rows = [rows0, rows1, rows2]
    gsem = [gs0, gs1, gs2]

    zrow = jnp.zeros((L,), jnp.float32)

    @pl.loop(0, 64)
    def _(i):
        for q in range(d // L):
            zbuf[i, pl.ds(q * L, L)] = zrow

    for j in range(10):
        pltpu.sync_copy(zbuf, acc_sh.at[pl.ds(s * NROW + j * 64, 64), :])
    plsc.subcore_barrier()

    def fetch(cc, p):
        be = w * EPT + cc * CHUNK
        pltpu.async_copy(src_hbm.at[pl.ds(be, CHUNK)], sb[p], isem)
        pltpu.async_copy(dst_hbm.at[pl.ds(be, CHUNK)], db[p], jsem)
        pltpu.make_async_copy(src_hbm.at[pl.ds(be, CHUNK)], sb[p],
                              isem).wait()
        pltpu.make_async_copy(dst_hbm.at[pl.ds(be, CHUNK)], db[p],
                              jsem).wait()
        pltpu.async_copy(g_hbm.at[sb[p]], rows[p], gsem[p])

    for p in range(3):
        fetch(p, p)

    @pl.loop(0, (NCHUNK + 2) // 3)
    def _(i):
        for p in range(3):
            cc = i * 3 + p

            @pl.when(cc < NCHUNK)
            def _(p=p, cc=cc):
                pltpu.make_async_copy(g_hbm.at[sb[p]], rows[p],
                                      gsem[p]).wait()
                pltpu.sync_copy(rows[p], acc_sh.at[db[p]], add=True)

            @pl.when(cc + 3 < NCHUNK)
            def _(p=p, cc=cc):
                fetch(cc + 3, p)

    plsc.subcore_barrier()
    for j in range(5):
        r0 = s * NROW + j * ZCH
        pltpu.sync_copy(acc_sh.at[pl.ds(r0, ZCH), :],
                        out_hbm.at[c, pl.ds(r0, ZCH), :])


def _make_add_call(d):
    @jax.jit
    def call(g, src, dst):
        return pl.kernel(
            functools.partial(_add_body, d),
            out_type=jax.ShapeDtypeStruct((NC, N_PAD, d), jnp.float32),
            mesh=_MESH,
            compiler_params=_SC_PARAMS if d % 128 == 0 else _SC_PARAMS_U,
            scratch_types=(
                [pltpu.VMEM((CHUNK,), jnp.int32)] * 6
                + [pltpu.VMEM((CHUNK, d), jnp.float32)] * 3
                + [pltpu.VMEM((64, d), jnp.float32),
                   pltpu.VMEM_SHARED((N_PAD, d), jnp.float32)]
                + [pltpu.SemaphoreType.DMA] * 5
            ),
        )(g, src, dst)

    return call


_add_call_128 = _make_add_call(D_H)
_add_call_32 = _make_add_call(D_H2)


def _bucket_body(asrc_hbm, adst_hbm, bsrc_hbm, bdst_hbm, cnt_hbm,
                 lsrc0, ldst0, lsrc1, ldst1, ssrc, sdst, cbuf,
                 la0, lb0, la1, lb1):
    w = _wid()
    base = w * OWN
    hi = base + OWN
    bb = w * CAPB

    lsrc = [lsrc0, lsrc1]
    ldst = [ldst0, ldst1]
    lsem = [la0, la1]
    dsem = [lb0, lb1]

    pltpu.sync_copy(asrc_hbm.at[pl.ds(0, BLK)], lsrc0)
    pltpu.sync_copy(adst_hbm.at[pl.ds(0, BLK)], ldst0)

    def process(blk, ls, ld, carry):
        off, nb = carry

        def grp(g, off):
            dv = ld[pl.ds(g * L, L)]
            sv = ls[pl.ds(g * L, L)]
            m = (dv >= base) & (dv < hi)
            plsc.store_compressed(sdst.at[pl.ds(off, L)], dv, mask=m)
            plsc.store_compressed(ssrc.at[pl.ds(off, L)], sv, mask=m)
            cntv = plsc.all_reduce_population_count(m)
            return off + jnp.squeeze(lax.slice(cntv, (0,), (1,)))

        off = lax.fori_loop(0, NGRP, grp, off, unroll=False)

        nunits = off // 80

        def flush(u, _):
            pltpu.sync_copy(ssrc.at[pl.ds(u * 80, 80)],
                            bsrc_hbm.at[pl.ds(bb + (nb + u) * 80, 80)])
            pltpu.sync_copy(sdst.at[pl.ds(u * 80, 80)],
                            bdst_hbm.at[pl.ds(bb + (nb + u) * 80, 80)])
            return 0

        lax.fori_loop(0, nunits, flush, 0, unroll=False)
        rem = off - nunits * 80
        for r in range(5):
            sv = ssrc[pl.ds(nunits * 80 + r * L, L)]
            dv = sdst[pl.ds(nunits * 80 + r * L, L)]
            ssrc[pl.ds(r * L, L)] = sv
            sdst[pl.ds(r * L, L)] = dv
        return (rem, nb + nunits)

    def rnd(i, carry):
        for p in range(2):
            blk = 2 * i + p

            @pl.when(blk + 1 < NBLK)
            def _():
                nxt = (blk + 1) * BLK
                pltpu.async_copy(asrc_hbm.at[pl.ds(nxt, BLK)], lsrc[1 - p],
                                 lsem[1 - p])
                pltpu.async_copy(adst_hbm.at[pl.ds(nxt, BLK)], ldst[1 - p],
                                 dsem[1 - p])

            @pl.when(blk > 0)
            def _():
                pltpu.make_async_copy(asrc_hbm.at[pl.ds(blk * BLK, BLK)],
                                      lsrc[p], lsem[p]).wait()
                pltpu.make_async_copy(adst_hbm.at[pl.ds(blk * BLK, BLK)],
                                      ldst[p], dsem[p]).wait()

            carry = process(blk, lsrc[p], ldst[p], carry)
        return carry

    off, nb = lax.fori_loop(0, NBLK // 2, rnd, (jnp.int32(0), jnp.int32(0)),
                            unroll=False)

    sent_s = jnp.full((L,), N, jnp.int32)
    sent_d = jnp.full((L,), base, jnp.int32)
    for r in range(5):
        ssrc[pl.ds(off + r * L, L)] = sent_s
        sdst[pl.ds(off + r * L, L)] = sent_d

    @pl.when(off > 0)
    def _():
        pltpu.sync_copy(ssrc.at[pl.ds(0, 80)],
                        bsrc_hbm.at[pl.ds(bb + nb * 80, 80)])
        pltpu.sync_copy(sdst.at[pl.ds(0, 80)],
                        bdst_hbm.at[pl.ds(bb + nb * 80, 80)])

    nbf = nb + (off > 0).astype(jnp.int32)
    cbuf[pl.ds(0, L)] = jnp.full((L,), 0, jnp.int32) + nbf
    pltpu.sync_copy(cbuf, cnt_hbm.at[pl.ds(w * L, L)])


@jax.jit
def _bucket_call(aj_src, aj_dst):
    return pl.kernel(
        _bucket_body,
        out_type=(
            jax.ShapeDtypeStruct((NW * CAPB,), jnp.int32),
            jax.ShapeDtypeStruct((NW * CAPB,), jnp.int32),
            jax.ShapeDtypeStruct((NW * L,), jnp.int32),
        ),
        mesh=_MESH,
        compiler_params=_SC_PARAMS,
        scratch_types=(
            [pltpu.VMEM((BLK,), jnp.int32)] * 4
            + [pltpu.VMEM((BLK + 96,), jnp.int32)] * 2
            + [pltpu.VMEM((L,), jnp.int32)]
            + [pltpu.SemaphoreType.DMA] * 4
        ),
    )(aj_src, aj_dst)


def _max_body(d, tab_hbm, bsrc_hbm, bdst_hbm, cnt_hbm, out_hbm,
              sb0, sb1, sb2, sb3, sb4, db0, db1, db2, db3, db4,
              rows0, rows1, rows2, rows3, rows4,
              acc, cbuf, gs0, gs1, gs2, gs3, gs4, isem, jsem):
    w = _wid()
    base = w * OWN
    bb = w * CAPB

    sb = [sb0, sb1, sb2, sb3, sb4]
    db = [db0, db1, db2, db3, db4]
    rows = [rows0, rows1, rows2, rows3, rows4]
    gsem = [gs0, gs1, gs2, gs3, gs4]

    ninf = jnp.full((L,), NEG, jnp.float32)

    @pl.loop(0, OWN * d // L)
    def _(i):
        acc[pl.ds(i * L, L)] = ninf

    pltpu.sync_copy(cnt_hbm.at[pl.ds(w * L, L)], cbuf)
    nb = jnp.max(cbuf[pl.ds(0, L)], axis=0)

    def fetch(cc, p):
        pltpu.async_copy(bsrc_hbm.at[pl.ds(bb + cc * 80, 80)], sb[p], isem)
        pltpu.async_copy(bdst_hbm.at[pl.ds(bb + cc * 80, 80)], db[p], jsem)
        pltpu.make_async_copy(bsrc_hbm.at[pl.ds(bb + cc * 80, 80)], sb[p],
                              isem).wait()
        pltpu.make_async_copy(bdst_hbm.at[pl.ds(bb + cc * 80, 80)], db[p],
                              jsem).wait()
        pltpu.async_copy(tab_hbm.at[sb[p]], rows[p], gsem[p])

    for p in range(5):
        @pl.when(p < nb)
        def _(p=p):
            fetch(p, p)

    def rmw(p, cc):
        pltpu.make_async_copy(tab_hbm.at[sb[p]], rows[p], gsem[p]).wait()

        def grp(g, _):
            offv = (db[p][pl.ds(g * L, L)] - base) * d
            dlos = [jnp.squeeze(lax.slice(offv, (k,), (k + 1,)))
                    for k in range(L)]
            for k in range(L):
                dlo = dlos[k]
                e = g * L + k
                avs = [acc[pl.ds(dlo + q * L, L)] for q in range(d // L)]
                rvs = [rows[p][e, pl.ds(q * L, L)] for q in range(d // L)]
                for q in range(d // L):
                    acc[pl.ds(dlo + q * L, L)] = jnp.maximum(avs[q], rvs[q])
            return 0

        lax.fori_loop(0, 5, grp, 0, unroll=False)

    def rnd(i, _):
        for p in range(5):
            cc = i * 5 + p

            @pl.when(cc < nb)
            def _(p=p, cc=cc):
                rmw(p, cc)

            @pl.when(cc + 5 < nb)
            def _(p=p, cc=cc):
                fetch(cc + 5, p)
        return 0

    lax.fori_loop(0, (nb + 4) // 5, rnd, 0, unroll=False)

    pltpu.sync_copy(acc, out_hbm.at[pl.ds(w * OWN * d, OWN * d)])


def _make_max_call(d):
    @jax.jit
    def call(tab, bsrc, bdst, bcnt):
        flat = pl.kernel(
            functools.partial(_max_body, d),
            out_type=jax.ShapeDtypeStruct((N_PAD * d,), jnp.float32),
            mesh=_MESH,
            compiler_params=_SC_PARAMS if d % 128 == 0 else _SC_PARAMS_U,
            scratch_types=(
                [pltpu.VMEM((80,), jnp.int32)] * 10
                + [pltpu.VMEM((80, d), jnp.float32)] * 5
                + [pltpu.VMEM((OWN * d,), jnp.float32),
                   pltpu.VMEM((L,), jnp.int32)]
                + [pltpu.SemaphoreType.DMA] * 7
            ),
        )(tab, bsrc, bdst, bcnt)
        return flat.reshape(N_PAD, d)

    return call


_max_call_128 = _make_max_call(D_H)
_max_call_32 = _make_max_call(D_H2)


def _dinv(deg_ref):
    total = 1.0 + deg_ref[0, :, 0:1] + deg_ref[1, :, 0:1]
    return lax.rsqrt(total)


def _tc1_body(x_ref, w1_ref, deg_ref, g_ref):
    h = jnp.dot(x_ref[...], w1_ref[...], preferred_element_type=jnp.float32)
    g_ref[...] = h * _dinv(deg_ref)


@jax.jit
def _tc1(x, W1, deg):
    return pl.pallas_call(
        _tc1_body,
        out_shape=jax.ShapeDtypeStruct((N, D_H), jnp.float32),
    )(x, W1, deg)


def _tc_mid_body(dh, p_ref, g_ref, deg_ref, we_ref, be_ref, b_ref,
                 a_ref, bv_ref):
    dinv = _dinv(deg_ref)
    h = jax.nn.relu(dinv * (p_ref[0] + p_ref[1] + g_ref[...]) + b_ref[...])
    wt = we_ref[0:dh, :]
    wb = we_ref[dh:2 * dh, :]
    a_ref[...] = jnp.dot(h, wt - wb, preferred_element_type=jnp.float32) \
        + be_ref[...]
    bv_ref[...] = jnp.dot(h, wb, preferred_element_type=jnp.float32)


def _make_tc_mid(dh):
    @jax.jit
    def call(p, g, deg, We, be, b):
        return pl.pallas_call(
            functools.partial(_tc_mid_body, dh),
            out_shape=(
                jax.ShapeDtypeStruct((N, dh), jnp.float32),
                jax.ShapeDtypeStruct((N, dh), jnp.float32),
            ),
        )(p, g, deg, We, be, b)

    return call


_tc2 = _make_tc_mid(D_H)
_tc4 = _make_tc_mid(D_H2)


def _tc3_body(m_ref, a_ref, deg_ref, w2_ref, g_ref):
    m = m_ref[...]
    h2 = jnp.where(m == NEG, 0.0, jax.nn.relu(a_ref[...] + m))
    g_ref[...] = jnp.dot(h2, w2_ref[...],
                         preferred_element_type=jnp.float32) * _dinv(deg_ref)


@jax.jit
def _tc3(m1, a1, deg, W2):
    return pl.pallas_call(
        _tc3_body,
        out_shape=jax.ShapeDtypeStruct((N, D_H2), jnp.float32),
    )(m1, a1, deg, W2)


def _tc5_body(m_ref, a_ref, wfc_ref, bfc_ref, batch_ref, out_ref):
    m = m_ref[...]
    h4 = jnp.where(m == NEG, 0.0, jax.nn.relu(a_ref[...] + m))
    y = jnp.dot(h4, wfc_ref[...], preferred_element_type=jnp.float32) \
        + bfc_ref[...]
    gid = batch_ref[...]
    onehot = (gid == lax.broadcasted_iota(jnp.int32, (1, NG), 1))
    onehot = onehot.astype(jnp.float32)
    s = jnp.sum(onehot * y, axis=0)
    cnt = jnp.sum(onehot, axis=0)
    out_ref[...] = (s / jnp.maximum(cnt, 1.0))[:, None]


@jax.jit
def _tc5(m2, a2, Wfc, bfc, batch2d):
    return pl.pallas_call(
        _tc5_body,
        out_shape=jax.ShapeDtypeStruct((NG, 1), jnp.float32),
    )(m2, a2, Wfc, bfc, batch2d)


@jax.jit
def kernel(x, edge_index, adj, batch, W1, b1, We1, be1, W2, b2, We2, be2,
           Wfc, bfc):
    ei_src = edge_index[0]
    ei_dst = edge_index[1]
    ei_dst3 = ei_dst.reshape(NW, NCHUNK, CHUNK)
    aj_src = adj[0]
    aj_dst = adj[1]

    deg = _deg_call(ei_dst3)[:, :N]
    bsrc, bdst, bcnt = _bucket_call(aj_src, aj_dst)

    g1 = _tc1(x, W1, deg)
    p1 = _add_call_128(g1, ei_src, ei_dst)[:, :N]
    a1, bv1 = _tc2(p1, g1, deg, We1, be1, b1)

    tab1 = jnp.concatenate([bv1, jnp.full((1, D_H), NEG, jnp.float32)], 0)
    m1 = _max_call_128(tab1, bsrc, bdst, bcnt)[:N]

    g2 = _tc3(m1, a1, deg, W2)
    p2 = _add_call_32(g2, ei_src, ei_dst)[:, :N]
    a2, bv2 = _tc4(p2, g2, deg, We2, be2, b2)

    tab2 = jnp.concatenate([bv2, jnp.full((1, D_H2), NEG, jnp.float32)], 0)
    m2 = _max_call_32(tab2, bsrc, bdst, bcnt)[:N]

    return _tc5(m2, a2, Wfc, bfc, batch.reshape(N, 1))
